# baseline (speedup 1.0000x reference)
import functools

import jax
import jax.numpy as jnp
from jax import lax
from jax.experimental import pallas as pl
from jax.experimental.pallas import tpu as pltpu
from jax.experimental.pallas import tpu_sc as plsc

N = 10000
D = 128
E = 320000

NC = 2
NS = 16
NW = NC * NS
C = 128
CB = 16
BLKS = 5
CHUNKS = CB * BLKS
E_PAD = CHUNKS * NW * C
N_ACC = 10112
ZROWS = 632
DEGW = 128
DEGO = 128

_mesh = functools.partial(
    plsc.VectorSubcoreMesh,
    core_axis_name="c", subcore_axis_name="s",
    num_cores=NC, num_subcores=NS,
)


ZBR = 40


def _zero_acc(zbuf, acc, sid, width):
    zero16 = jnp.zeros((16,), dtype=jnp.float32)

    def fill(i, _):
        zbuf[i // (width // 16), pl.ds((i % (width // 16)) * 16, 16)] = zero16
        return 0
    lax.fori_loop(0, ZBR * (width // 16), fill, 0)

    def zcopy(q, _):
        pltpu.sync_copy(zbuf, acc.at[pl.ds(sid * ZROWS + q * ZBR, ZBR)])
        return 0
    lax.fori_loop(0, ZROWS // ZBR, zcopy, 0)
    pltpu.sync_copy(zbuf.at[pl.ds(0, ZROWS - ZBR * (ZROWS // ZBR))],
                    acc.at[pl.ds(sid * ZROWS + ZBR * (ZROWS // ZBR),
                                 ZROWS - ZBR * (ZROWS // ZBR))])


def _sc_deg_body(dst_ref, degp_ref, didx_b, ones_v, zbuf, acc, sem):
    cid = lax.axis_index("c")
    sid = lax.axis_index("s")
    wid = cid * NS + sid
    one16 = jnp.full((16,), 1.0, dtype=jnp.float32)

    def fillo(i, _):
        ones_v[i // (DEGW // 16), pl.ds((i % (DEGW // 16)) * 16, 16)] = one16
        return 0
    lax.fori_loop(0, C * (DEGW // 16), fillo, 0)
    _zero_acc(zbuf, acc, sid, DEGW)
    plsc.subcore_barrier()

    def block(b, _):
        base = wid * CHUNKS + b * CB
        pltpu.sync_copy(dst_ref.at[pl.ds(base, CB)], didx_b)

        def fire(k, _):
            pltpu.async_copy(ones_v, acc.at[didx_b.at[k]], sem, add=True)
            return 0
        lax.fori_loop(0, CB, fire, 0)

        def drain(k, _):
            pltpu.make_async_copy(ones_v, acc.at[didx_b.at[k]], sem).wait()
            return 0
        lax.fori_loop(0, CB, drain, 0)
        return 0
    lax.fori_loop(0, BLKS, block, 0)
    plsc.subcore_barrier()

    pltpu.sync_copy(acc.at[pl.ds(sid * ZROWS, ZROWS)],
                    degp_ref.at[cid, pl.ds(sid * ZROWS, ZROWS)])


_sc_deg = pl.kernel(
    _sc_deg_body,
    out_type=jax.ShapeDtypeStruct((NC, N_ACC, DEGO), jnp.float32),
    mesh=_mesh(),
    scratch_types=[
        pltpu.VMEM((CB, C), jnp.int32),
        pltpu.VMEM((C, DEGW), jnp.float32),
        pltpu.VMEM((ZBR, DEGW), jnp.float32),
        pltpu.VMEM_SHARED((N_ACC, DEGW), jnp.float32),
        pltpu.SemaphoreType.DMA,
    ],
)


def _sc_scatter_body(src_ref, dst_ref, hp_ref, sp_ref,
                     sidx_b, didx_b, rows_a, rows_b, zbuf, acc,
                     sem_a, sem_b, ssem_a, ssem_b):
    cid = lax.axis_index("c")
    sid = lax.axis_index("s")
    wid = cid * NS + sid
    _zero_acc(zbuf, acc, sid, D)
    plsc.subcore_barrier()

    def G(k, rbuf, gs):
        pltpu.async_copy(hp_ref.at[sidx_b.at[k]], rbuf, gs)

    def WG(k, rbuf, gs):
        pltpu.make_async_copy(hp_ref.at[sidx_b.at[k]], rbuf, gs).wait()

    def S(k, rbuf, ss):
        pltpu.async_copy(rbuf, acc.at[didx_b.at[k]], ss, add=True)

    def WS(k, rbuf, ss):
        pltpu.make_async_copy(rbuf, acc.at[didx_b.at[k]], ss).wait()

    def block(b, _):
        base = wid * CHUNKS + b * CB
        pltpu.sync_copy(src_ref.at[pl.ds(base, CB)], sidx_b)
        pltpu.sync_copy(dst_ref.at[pl.ds(base, CB)], didx_b)
        G(0, rows_a, sem_a)
        G(1, rows_b, sem_b)

        def pair(t, _):
            k = 2 * t
            WG(k, rows_a, sem_a)
            S(k, rows_a, ssem_a)
            WG(k + 1, rows_b, sem_b)
            S(k + 1, rows_b, ssem_b)
            WS(k, rows_a, ssem_a)
            G(k + 2, rows_a, sem_a)
            WS(k + 1, rows_b, ssem_b)
            G(k + 3, rows_b, sem_b)
            return 0
        lax.fori_loop(0, CB // 2 - 2, pair, 0)
        k = CB - 4
        WG(k, rows_a, sem_a)
        S(k, rows_a, ssem_a)
        WG(k + 1, rows_b, sem_b)
        S(k + 1, rows_b, ssem_b)
        WS(k, rows_a, ssem_a)
        G(k + 2, rows_a, sem_a)
        WS(k + 1, rows_b, ssem_b)
        G(k + 3, rows_b, sem_b)
        WG(k + 2, rows_a, sem_a)
        S(k + 2, rows_a, ssem_a)
        WG(k + 3, rows_b, sem_b)
        S(k + 3, rows_b, ssem_b)
        WS(k + 2, rows_a, ssem_a)
        WS(k + 3, rows_b, ssem_b)
        return 0
    lax.fori_loop(0, BLKS, block, 0)
    plsc.subcore_barrier()

    pltpu.sync_copy(acc.at[pl.ds(sid * ZROWS, ZROWS)],
                    sp_ref.at[cid, pl.ds(sid * ZROWS, ZROWS)])


_sc_scatter = pl.kernel(
    _sc_scatter_body,
    out_type=jax.ShapeDtypeStruct((NC, N_ACC, D), jnp.float32),
    mesh=_mesh(),
    scratch_types=[
        pltpu.VMEM((CB, C), jnp.int32),
        pltpu.VMEM((CB, C), jnp.int32),
        pltpu.VMEM((C, D), jnp.float32),
        pltpu.VMEM((C, D), jnp.float32),
        pltpu.VMEM((ZBR, D), jnp.float32),
        pltpu.VMEM_SHARED((N_ACC, D), jnp.float32),
        pltpu.SemaphoreType.DMA,
        pltpu.SemaphoreType.DMA,
        pltpu.SemaphoreType.DMA,
        pltpu.SemaphoreType.DMA,
    ],
)

_BLK = 1000
_GRID = N // _BLK


def _tc_hp_body(x_ref, w_ref, degp_ref, hp_ref):
    deg = 1.0 + degp_ref[0, :, 0:1] + degp_ref[1, :, 0:1]
    dinv = lax.rsqrt(deg)
    h = jnp.dot(x_ref[...], w_ref[...], preferred_element_type=jnp.float32)
    hp_ref[...] = h * dinv


def _tc_hp(x, W, degp):
    return pl.pallas_call(
        _tc_hp_body,
        grid=(_GRID,),
        in_specs=[
            pl.BlockSpec((_BLK, D), lambda i: (i, 0)),
            pl.BlockSpec((D, D), lambda i: (0, 0)),
            pl.BlockSpec((NC, _BLK, DEGO), lambda i: (0, i, 0)),
        ],
        out_specs=pl.BlockSpec((_BLK, D), lambda i: (i, 0)),
        out_shape=jax.ShapeDtypeStruct((N, D), jnp.float32),
    )(x, W, degp)


def _tc_gru_body(sp_ref, degp_ref, hp_ref, xp_ref, wih_ref, whh_ref,
                 bih_ref, bhh_ref, b_ref, out_ref):
    deg = 1.0 + degp_ref[0, :, 0:1] + degp_ref[1, :, 0:1]
    dinv = lax.rsqrt(deg)
    S = sp_ref[0] + sp_ref[1]
    hp = hp_ref[...]
    gcn = dinv * (S + hp) + b_ref[...]
    xp = xp_ref[...]
    gi = jnp.dot(gcn, wih_ref[...], preferred_element_type=jnp.float32) + bih_ref[...]
    gh = jnp.dot(xp, whh_ref[...], preferred_element_type=jnp.float32) + bhh_ref[...]
    r = jax.nn.sigmoid(gi[:, :D] + gh[:, :D])
    z = jax.nn.sigmoid(gi[:, D:2 * D] + gh[:, D:2 * D])
    n = jnp.tanh(gi[:, 2 * D:] + r * gh[:, 2 * D:])
    out_ref[...] = (1.0 - z) * n + z * xp


def _tc_gru(sp, degp, hp, x_prev, W_ih_T, W_hh_T, b_ih, b_hh, b):
    return pl.pallas_call(
        _tc_gru_body,
        grid=(_GRID,),
        in_specs=[
            pl.BlockSpec((NC, _BLK, D), lambda i: (0, i, 0)),
            pl.BlockSpec((NC, _BLK, DEGO), lambda i: (0, i, 0)),
            pl.BlockSpec((_BLK, D), lambda i: (i, 0)),
            pl.BlockSpec((_BLK, D), lambda i: (i, 0)),
            pl.BlockSpec((D, 3 * D), lambda i: (0, 0)),
            pl.BlockSpec((D, 3 * D), lambda i: (0, 0)),
            pl.BlockSpec((1, 3 * D), lambda i: (0, 0)),
            pl.BlockSpec((1, 3 * D), lambda i: (0, 0)),
            pl.BlockSpec((1, D), lambda i: (0, 0)),
        ],
        out_specs=pl.BlockSpec((_BLK, D), lambda i: (i, 0)),
        out_shape=jax.ShapeDtypeStruct((N, D), jnp.float32),
    )(sp, degp, hp, x_prev, W_ih_T, W_hh_T, b_ih, b_hh, b)


def kernel(x, edge_index, x_prev, W, b, W_ih, W_hh, b_ih, b_hh):
    src = edge_index[0].astype(jnp.int32)
    dst = edge_index[1].astype(jnp.int32)
    pad = jnp.arange(E_PAD - E, dtype=jnp.int32)
    src = jnp.concatenate([src, (pad * 997) % N])
    dst = jnp.concatenate([dst, N + pad % (N_ACC - N)])
    src2d = src.reshape(E_PAD // C, C)
    dst2d = dst.reshape(E_PAD // C, C)

    degp = _sc_deg(dst2d)
    hp = _tc_hp(x, W, degp)
    sp = _sc_scatter(src2d, dst2d, hp)
    return _tc_gru(sp, degp, hp, x_prev,
                   W_ih.T, W_hh.T,
                   b_ih.reshape(1, 3 * D), b_hh.reshape(1, 3 * D),
                   b.reshape(1, D))


# speedup vs baseline: 1.1356x; 1.1356x over previous
import functools

import jax
import jax.numpy as jnp
from jax import lax
from jax.experimental import pallas as pl
from jax.experimental.pallas import tpu as pltpu
from jax.experimental.pallas import tpu_sc as plsc

N = 10000
D = 128
E = 320000

NC = 2
NS = 16
NW = NC * NS
C = 128
CB = 16
BLKS = 5
CHUNKS = CB * BLKS
E_PAD = CHUNKS * NW * C
N_ACC = 10112
ZROWS = 632
DEGW = 128
DEGO = 128

_mesh = functools.partial(
    plsc.VectorSubcoreMesh,
    core_axis_name="c", subcore_axis_name="s",
    num_cores=NC, num_subcores=NS,
)


ZBR = 40


def _zero_acc(zbuf, acc, sid, width):
    zero16 = jnp.zeros((16,), dtype=jnp.float32)

    def fill(i, _):
        zbuf[i // (width // 16), pl.ds((i % (width // 16)) * 16, 16)] = zero16
        return 0
    lax.fori_loop(0, ZBR * (width // 16), fill, 0)

    def zcopy(q, _):
        pltpu.sync_copy(zbuf, acc.at[pl.ds(sid * ZROWS + q * ZBR, ZBR)])
        return 0
    lax.fori_loop(0, ZROWS // ZBR, zcopy, 0)
    pltpu.sync_copy(zbuf.at[pl.ds(0, ZROWS - ZBR * (ZROWS // ZBR))],
                    acc.at[pl.ds(sid * ZROWS + ZBR * (ZROWS // ZBR),
                                 ZROWS - ZBR * (ZROWS // ZBR))])


def _sc_deg_body(dst_ref, degp_ref, didx_b, ones_v, zbuf, acc, sem):
    cid = lax.axis_index("c")
    sid = lax.axis_index("s")
    wid = cid * NS + sid
    one16 = jnp.full((16,), 1.0, dtype=jnp.float32)

    def fillo(i, _):
        ones_v[i // (DEGW // 16), pl.ds((i % (DEGW // 16)) * 16, 16)] = one16
        return 0
    lax.fori_loop(0, C * (DEGW // 16), fillo, 0)
    _zero_acc(zbuf, acc, sid, DEGW)
    plsc.subcore_barrier()

    def block(b, _):
        base = wid * CHUNKS + b * CB
        pltpu.sync_copy(dst_ref.at[pl.ds(base, CB)], didx_b)

        def fire(k, _):
            pltpu.async_copy(ones_v, acc.at[didx_b.at[k]], sem, add=True)
            return 0
        lax.fori_loop(0, CB, fire, 0)

        def drain(k, _):
            pltpu.make_async_copy(ones_v, acc.at[didx_b.at[k]], sem).wait()
            return 0
        lax.fori_loop(0, CB, drain, 0)
        return 0
    lax.fori_loop(0, BLKS, block, 0)
    plsc.subcore_barrier()

    pltpu.sync_copy(acc.at[pl.ds(sid * ZROWS, ZROWS)],
                    degp_ref.at[cid, pl.ds(sid * ZROWS, ZROWS)])


_sc_deg = pl.kernel(
    _sc_deg_body,
    out_type=jax.ShapeDtypeStruct((NC, N_ACC, DEGO), jnp.float32),
    mesh=_mesh(),
    scratch_types=[
        pltpu.VMEM((CB, C), jnp.int32),
        pltpu.VMEM((C, DEGW), jnp.float32),
        pltpu.VMEM((ZBR, DEGW), jnp.float32),
        pltpu.VMEM_SHARED((N_ACC, DEGW), jnp.float32),
        pltpu.SemaphoreType.DMA,
    ],
)


def _sc_scatter_body(src_ref, dst_ref, hp_ref, sp_ref,
                     sidx_b, didx_b, rows_a, rows_b, zbuf, acc,
                     sem_a, sem_b):
    cid = lax.axis_index("c")
    sid = lax.axis_index("s")
    wid = cid * NS + sid
    _zero_acc(zbuf, acc, sid, D)
    plsc.subcore_barrier()

    def gather(k, rbuf, s):
        pltpu.async_copy(hp_ref.at[sidx_b.at[k]], rbuf, s)

    def scat(k, rbuf, s):
        pltpu.make_async_copy(hp_ref.at[sidx_b.at[k]], rbuf, s).wait()
        pltpu.sync_copy(rbuf, acc.at[didx_b.at[k]], add=True)

    def block(b, _):
        base = wid * CHUNKS + b * CB
        pltpu.sync_copy(src_ref.at[pl.ds(base, CB)], sidx_b)
        pltpu.sync_copy(dst_ref.at[pl.ds(base, CB)], didx_b)
        gather(0, rows_a, sem_a)

        def pair(jj, _):
            gather(2 * jj + 1, rows_b, sem_b)
            scat(2 * jj, rows_a, sem_a)
            gather(2 * jj + 2, rows_a, sem_a)
            scat(2 * jj + 1, rows_b, sem_b)
            return 0
        lax.fori_loop(0, CB // 2 - 1, pair, 0)
        gather(CB - 1, rows_b, sem_b)
        scat(CB - 2, rows_a, sem_a)
        scat(CB - 1, rows_b, sem_b)
        return 0
    lax.fori_loop(0, BLKS, block, 0)
    plsc.subcore_barrier()

    pltpu.sync_copy(acc.at[pl.ds(sid * ZROWS, ZROWS)],
                    sp_ref.at[cid, pl.ds(sid * ZROWS, ZROWS)])


_sc_scatter = pl.kernel(
    _sc_scatter_body,
    out_type=jax.ShapeDtypeStruct((NC, N_ACC, D), jnp.float32),
    mesh=_mesh(),
    scratch_types=[
        pltpu.VMEM((CB, C), jnp.int32),
        pltpu.VMEM((CB, C), jnp.int32),
        pltpu.VMEM((C, D), jnp.float32),
        pltpu.VMEM((C, D), jnp.float32),
        pltpu.VMEM((ZBR, D), jnp.float32),
        pltpu.VMEM_SHARED((N_ACC, D), jnp.float32),
        pltpu.SemaphoreType.DMA,
        pltpu.SemaphoreType.DMA,
    ],
)

_BLK = 1000
_GRID = N // _BLK


def _tc_h_body(x_ref, w_ref, h_ref):
    h_ref[...] = jnp.dot(x_ref[...], w_ref[...],
                         preferred_element_type=jnp.float32)


def _tc_h(x, W):
    return pl.pallas_call(
        _tc_h_body,
        grid=(_GRID,),
        in_specs=[
            pl.BlockSpec((_BLK, D), lambda i: (i, 0)),
            pl.BlockSpec((D, D), lambda i: (0, 0)),
        ],
        out_specs=pl.BlockSpec((_BLK, D), lambda i: (i, 0)),
        out_shape=jax.ShapeDtypeStruct((N, D), jnp.float32),
    )(x, W)


def _tc_hp_body(h_ref, degp_ref, hp_ref):
    deg = 1.0 + degp_ref[0, :, 0:1] + degp_ref[1, :, 0:1]
    dinv = lax.rsqrt(deg)
    hp_ref[...] = h_ref[...] * dinv


def _tc_hp(h, degp):
    return pl.pallas_call(
        _tc_hp_body,
        grid=(_GRID,),
        in_specs=[
            pl.BlockSpec((_BLK, D), lambda i: (i, 0)),
            pl.BlockSpec((NC, _BLK, DEGO), lambda i: (0, i, 0)),
        ],
        out_specs=pl.BlockSpec((_BLK, D), lambda i: (i, 0)),
        out_shape=jax.ShapeDtypeStruct((N, D), jnp.float32),
    )(h, degp)


def _tc_gru_body(sp_ref, degp_ref, hp_ref, xp_ref, wih_ref, whh_ref,
                 bih_ref, bhh_ref, b_ref, out_ref):
    deg = 1.0 + degp_ref[0, :, 0:1] + degp_ref[1, :, 0:1]
    dinv = lax.rsqrt(deg)
    S = sp_ref[0] + sp_ref[1]
    hp = hp_ref[...]
    gcn = dinv * (S + hp) + b_ref[...]
    xp = xp_ref[...]
    gi = jnp.dot(gcn, wih_ref[...], preferred_element_type=jnp.float32) + bih_ref[...]
    gh = jnp.dot(xp, whh_ref[...], preferred_element_type=jnp.float32) + bhh_ref[...]
    r = jax.nn.sigmoid(gi[:, :D] + gh[:, :D])
    z = jax.nn.sigmoid(gi[:, D:2 * D] + gh[:, D:2 * D])
    n = jnp.tanh(gi[:, 2 * D:] + r * gh[:, 2 * D:])
    out_ref[...] = (1.0 - z) * n + z * xp


def _tc_gru(sp, degp, hp, x_prev, W_ih_T, W_hh_T, b_ih, b_hh, b):
    return pl.pallas_call(
        _tc_gru_body,
        grid=(_GRID,),
        in_specs=[
            pl.BlockSpec((NC, _BLK, D), lambda i: (0, i, 0)),
            pl.BlockSpec((NC, _BLK, DEGO), lambda i: (0, i, 0)),
            pl.BlockSpec((_BLK, D), lambda i: (i, 0)),
            pl.BlockSpec((_BLK, D), lambda i: (i, 0)),
            pl.BlockSpec((D, 3 * D), lambda i: (0, 0)),
            pl.BlockSpec((D, 3 * D), lambda i: (0, 0)),
            pl.BlockSpec((1, 3 * D), lambda i: (0, 0)),
            pl.BlockSpec((1, 3 * D), lambda i: (0, 0)),
            pl.BlockSpec((1, D), lambda i: (0, 0)),
        ],
        out_specs=pl.BlockSpec((_BLK, D), lambda i: (i, 0)),
        out_shape=jax.ShapeDtypeStruct((N, D), jnp.float32),
    )(sp, degp, hp, x_prev, W_ih_T, W_hh_T, b_ih, b_hh, b)


def kernel(x, edge_index, x_prev, W, b, W_ih, W_hh, b_ih, b_hh):
    src = edge_index[0].astype(jnp.int32)
    dst = edge_index[1].astype(jnp.int32)
    pad = jnp.arange(E_PAD - E, dtype=jnp.int32)
    src = jnp.concatenate([src, (pad * 997) % N])
    dst = jnp.concatenate([dst, N + pad % (N_ACC - N)])
    src2d = src.reshape(E_PAD // C, C)
    dst2d = dst.reshape(E_PAD // C, C)

    h = _tc_h(x, W)
    degp = _sc_deg(dst2d)
    hp = _tc_hp(h, degp)
    sp = _sc_scatter(src2d, dst2d, hp)
    return _tc_gru(sp, degp, hp, x_prev,
                   W_ih.T, W_hh.T,
                   b_ih.reshape(1, 3 * D), b_hh.reshape(1, 3 * D),
                   b.reshape(1, D))


# speedup vs baseline: 1.1809x; 1.0399x over previous
import functools

import jax
import jax.numpy as jnp
from jax import lax
from jax.experimental import pallas as pl
from jax.experimental.pallas import tpu as pltpu
from jax.experimental.pallas import tpu_sc as plsc

N = 10000
D = 128
E = 320000

NC = 2
NS = 16
NW = NC * NS
C = 128
CB = 40
BLKS = 2
CHUNKS = CB * BLKS
E_PAD = CHUNKS * NW * C
N_ACC = 10112
ZROWS = 632
DEGW = 128
DEGO = 128

_mesh = functools.partial(
    plsc.VectorSubcoreMesh,
    core_axis_name="c", subcore_axis_name="s",
    num_cores=NC, num_subcores=NS,
)


ZBR = 40


def _zero_acc(zbuf, acc, sid, width):
    zero16 = jnp.zeros((16,), dtype=jnp.float32)

    def fill(i, _):
        zbuf[i // (width // 16), pl.ds((i % (width // 16)) * 16, 16)] = zero16
        return 0
    lax.fori_loop(0, ZBR * (width // 16), fill, 0)

    def zcopy(q, _):
        pltpu.sync_copy(zbuf, acc.at[pl.ds(sid * ZROWS + q * ZBR, ZBR)])
        return 0
    lax.fori_loop(0, ZROWS // ZBR, zcopy, 0)
    pltpu.sync_copy(zbuf.at[pl.ds(0, ZROWS - ZBR * (ZROWS // ZBR))],
                    acc.at[pl.ds(sid * ZROWS + ZBR * (ZROWS // ZBR),
                                 ZROWS - ZBR * (ZROWS // ZBR))])


def _sc_deg_body(dst_ref, degp_ref, didx_b, ones_v, zbuf, acc, sem):
    cid = lax.axis_index("c")
    sid = lax.axis_index("s")
    wid = cid * NS + sid
    one16 = jnp.full((16,), 1.0, dtype=jnp.float32)

    def fillo(i, _):
        ones_v[i // (DEGW // 16), pl.ds((i % (DEGW // 16)) * 16, 16)] = one16
        return 0
    lax.fori_loop(0, C * (DEGW // 16), fillo, 0)
    _zero_acc(zbuf, acc, sid, DEGW)
    plsc.subcore_barrier()

    def block(b, _):
        base = wid * CHUNKS + b * CB
        pltpu.sync_copy(dst_ref.at[pl.ds(base, CB)], didx_b)

        def fire(k, _):
            pltpu.async_copy(ones_v, acc.at[didx_b.at[k]], sem, add=True)
            return 0
        lax.fori_loop(0, CB, fire, 0)

        def drain(k, _):
            pltpu.make_async_copy(ones_v, acc.at[didx_b.at[k]], sem).wait()
            return 0
        lax.fori_loop(0, CB, drain, 0)
        return 0
    lax.fori_loop(0, BLKS, block, 0)
    plsc.subcore_barrier()

    pltpu.sync_copy(acc.at[pl.ds(sid * ZROWS, ZROWS)],
                    degp_ref.at[cid, pl.ds(sid * ZROWS, ZROWS)])


_sc_deg = pl.kernel(
    _sc_deg_body,
    out_type=jax.ShapeDtypeStruct((NC, N_ACC, DEGO), jnp.float32),
    mesh=_mesh(),
    scratch_types=[
        pltpu.VMEM((CB, C), jnp.int32),
        pltpu.VMEM((C, DEGW), jnp.float32),
        pltpu.VMEM((ZBR, DEGW), jnp.float32),
        pltpu.VMEM_SHARED((N_ACC, DEGW), jnp.float32),
        pltpu.SemaphoreType.DMA,
    ],
)


def _sc_scatter_body(src_ref, dst_ref, hp_ref, sp_ref,
                     sidx_b, didx_b, rows_a, rows_b, zbuf, acc,
                     sem_a, sem_b):
    cid = lax.axis_index("c")
    sid = lax.axis_index("s")
    wid = cid * NS + sid
    _zero_acc(zbuf, acc, sid, D)
    plsc.subcore_barrier()

    def gather(k, rbuf, s):
        pltpu.async_copy(hp_ref.at[sidx_b.at[k]], rbuf, s)

    def scat(k, rbuf, s):
        pltpu.make_async_copy(hp_ref.at[sidx_b.at[k]], rbuf, s).wait()
        pltpu.sync_copy(rbuf, acc.at[didx_b.at[k]], add=True)

    def block(b, _):
        base = wid * CHUNKS + b * CB
        pltpu.sync_copy(src_ref.at[pl.ds(base, CB)], sidx_b)
        pltpu.sync_copy(dst_ref.at[pl.ds(base, CB)], didx_b)
        gather(0, rows_a, sem_a)

        def pair(jj, _):
            gather(2 * jj + 1, rows_b, sem_b)
            scat(2 * jj, rows_a, sem_a)
            gather(2 * jj + 2, rows_a, sem_a)
            scat(2 * jj + 1, rows_b, sem_b)
            return 0
        lax.fori_loop(0, CB // 2 - 1, pair, 0)
        gather(CB - 1, rows_b, sem_b)
        scat(CB - 2, rows_a, sem_a)
        scat(CB - 1, rows_b, sem_b)
        return 0
    lax.fori_loop(0, BLKS, block, 0)
    plsc.subcore_barrier()

    pltpu.sync_copy(acc.at[pl.ds(sid * ZROWS, ZROWS)],
                    sp_ref.at[cid, pl.ds(sid * ZROWS, ZROWS)])


_sc_scatter = pl.kernel(
    _sc_scatter_body,
    out_type=jax.ShapeDtypeStruct((NC, N_ACC, D), jnp.float32),
    mesh=_mesh(),
    scratch_types=[
        pltpu.VMEM((CB, C), jnp.int32),
        pltpu.VMEM((CB, C), jnp.int32),
        pltpu.VMEM((C, D), jnp.float32),
        pltpu.VMEM((C, D), jnp.float32),
        pltpu.VMEM((ZBR, D), jnp.float32),
        pltpu.VMEM_SHARED((N_ACC, D), jnp.float32),
        pltpu.SemaphoreType.DMA,
        pltpu.SemaphoreType.DMA,
    ],
)

_BLK = 1000
_GRID = N // _BLK


def _tc_h_body(x_ref, w_ref, h_ref):
    h_ref[...] = jnp.dot(x_ref[...], w_ref[...],
                         preferred_element_type=jnp.float32)


def _tc_h(x, W):
    return pl.pallas_call(
        _tc_h_body,
        grid=(_GRID,),
        in_specs=[
            pl.BlockSpec((_BLK, D), lambda i: (i, 0)),
            pl.BlockSpec((D, D), lambda i: (0, 0)),
        ],
        out_specs=pl.BlockSpec((_BLK, D), lambda i: (i, 0)),
        out_shape=jax.ShapeDtypeStruct((N, D), jnp.float32),
    )(x, W)


def _tc_hp_body(h_ref, degp_ref, hp_ref):
    deg = 1.0 + degp_ref[0, :, 0:1] + degp_ref[1, :, 0:1]
    dinv = lax.rsqrt(deg)
    hp_ref[...] = h_ref[...] * dinv


def _tc_hp(h, degp):
    return pl.pallas_call(
        _tc_hp_body,
        grid=(_GRID,),
        in_specs=[
            pl.BlockSpec((_BLK, D), lambda i: (i, 0)),
            pl.BlockSpec((NC, _BLK, DEGO), lambda i: (0, i, 0)),
        ],
        out_specs=pl.BlockSpec((_BLK, D), lambda i: (i, 0)),
        out_shape=jax.ShapeDtypeStruct((N, D), jnp.float32),
    )(h, degp)


def _tc_gru_body(sp_ref, degp_ref, hp_ref, xp_ref, wih_ref, whh_ref,
                 bih_ref, bhh_ref, b_ref, out_ref):
    deg = 1.0 + degp_ref[0, :, 0:1] + degp_ref[1, :, 0:1]
    dinv = lax.rsqrt(deg)
    S = sp_ref[0] + sp_ref[1]
    hp = hp_ref[...]
    gcn = dinv * (S + hp) + b_ref[...]
    xp = xp_ref[...]
    gi = jnp.dot(gcn, wih_ref[...], preferred_element_type=jnp.float32) + bih_ref[...]
    gh = jnp.dot(xp, whh_ref[...], preferred_element_type=jnp.float32) + bhh_ref[...]
    r = jax.nn.sigmoid(gi[:, :D] + gh[:, :D])
    z = jax.nn.sigmoid(gi[:, D:2 * D] + gh[:, D:2 * D])
    n = jnp.tanh(gi[:, 2 * D:] + r * gh[:, 2 * D:])
    out_ref[...] = (1.0 - z) * n + z * xp


def _tc_gru(sp, degp, hp, x_prev, W_ih_T, W_hh_T, b_ih, b_hh, b):
    return pl.pallas_call(
        _tc_gru_body,
        grid=(_GRID,),
        in_specs=[
            pl.BlockSpec((NC, _BLK, D), lambda i: (0, i, 0)),
            pl.BlockSpec((NC, _BLK, DEGO), lambda i: (0, i, 0)),
            pl.BlockSpec((_BLK, D), lambda i: (i, 0)),
            pl.BlockSpec((_BLK, D), lambda i: (i, 0)),
            pl.BlockSpec((D, 3 * D), lambda i: (0, 0)),
            pl.BlockSpec((D, 3 * D), lambda i: (0, 0)),
            pl.BlockSpec((1, 3 * D), lambda i: (0, 0)),
            pl.BlockSpec((1, 3 * D), lambda i: (0, 0)),
            pl.BlockSpec((1, D), lambda i: (0, 0)),
        ],
        out_specs=pl.BlockSpec((_BLK, D), lambda i: (i, 0)),
        out_shape=jax.ShapeDtypeStruct((N, D), jnp.float32),
    )(sp, degp, hp, x_prev, W_ih_T, W_hh_T, b_ih, b_hh, b)


def kernel(x, edge_index, x_prev, W, b, W_ih, W_hh, b_ih, b_hh):
    src = edge_index[0].astype(jnp.int32)
    dst = edge_index[1].astype(jnp.int32)
    pad = jnp.arange(E_PAD - E, dtype=jnp.int32)
    src = jnp.concatenate([src, (pad * 997) % N])
    dst = jnp.concatenate([dst, N + pad % (N_ACC - N)])
    src2d = src.reshape(E_PAD // C, C)
    dst2d = dst.reshape(E_PAD // C, C)

    h = _tc_h(x, W)
    degp = _sc_deg(dst2d)
    hp = _tc_hp(h, degp)
    sp = _sc_scatter(src2d, dst2d, hp)
    return _tc_gru(sp, degp, hp, x_prev,
                   W_ih.T, W_hh.T,
                   b_ih.reshape(1, 3 * D), b_hh.reshape(1, 3 * D),
                   b.reshape(1, D))
